# baseline (speedup 1.0000x reference)
import functools

import jax
import jax.numpy as jnp
from jax import lax
from jax.experimental import pallas as pl
from jax.experimental.pallas import tpu as pltpu
from jax.experimental.pallas import tpu_sc as plsc

_N = 1_000_000
_ON = 8192
_OFF = 8192
_NW = 32
_BPW = _ON // _NW

_mesh = plsc.VectorSubcoreMesh(core_axis_name="c", subcore_axis_name="s")


_EPW = _BPW * 3


@functools.partial(
    pl.kernel,
    mesh=_mesh,
    out_type=[
        jax.ShapeDtypeStruct((_ON * 3,), jnp.float32),
        jax.ShapeDtypeStruct((_ON * 3,), jnp.float32),
    ],
    scratch_types=[
        pltpu.VMEM((_EPW,), jnp.int32),
        pltpu.VMEM((_EPW,), jnp.float32),
        pltpu.VMEM((_EPW,), jnp.float32),
        pltpu.SemaphoreType.DMA,
        pltpu.SemaphoreType.DMA,
    ],
)
def _sc_gather(coords_hbm, normals_hbm, idx3_hbm, outc_hbm, outn_hbm,
               idx_v, rc_v, rn_v, sem_c, sem_n):
    wid = lax.axis_index("s") * 2 + lax.axis_index("c")
    base = wid * _EPW
    pltpu.sync_copy(idx3_hbm.at[pl.ds(base, _EPW)], idx_v)
    cpy_c = pltpu.async_copy(coords_hbm.at[idx_v], rc_v, sem_c)
    cpy_n = pltpu.async_copy(normals_hbm.at[idx_v], rn_v, sem_n)
    cpy_c.wait()
    cpy_n.wait()
    pltpu.sync_copy(rc_v, outc_hbm.at[pl.ds(base, _EPW)])
    pltpu.sync_copy(rn_v, outn_hbm.at[pl.ds(base, _EPW)])


def kernel(coords, normals, key):
    k = jax.random.key(key)
    idx = jax.random.choice(k, coords.shape[0], shape=(_ON,), replace=False)
    off_coords = jax.random.uniform(k, shape=(_OFF, 3), minval=-1.0, maxval=1.0)
    idx3 = (idx.astype(jnp.int32)[:, None] * 3
            + jnp.arange(3, dtype=jnp.int32)[None, :]).reshape(-1)
    gc, gn = _sc_gather(coords.reshape(-1), normals.reshape(-1), idx3)
    gc = gc.reshape(_ON, 3)
    gn = gn.reshape(_ON, 3)
    out_coords = jnp.concatenate([gc, off_coords], axis=0)
    out_normals = jnp.concatenate([gn, jnp.full((_OFF, 3), -1.0, jnp.float32)],
                                  axis=0)
    sdf = jnp.concatenate([jnp.zeros((_ON, 1), jnp.float32),
                           jnp.full((_OFF, 1), -1.0, jnp.float32)], axis=0)
    return out_coords, out_normals, sdf


# speedup vs baseline: 2.6753x; 2.6753x over previous
import functools

import jax
import jax.numpy as jnp
from jax import lax
from jax.experimental import pallas as pl
from jax.experimental.pallas import tpu as pltpu
from jax.experimental.pallas import tpu_sc as plsc

_N = 1_000_000
_ON = 8192
_OFF = 8192
_NW = 32
_BPW = _ON // _NW

_mesh = plsc.VectorSubcoreMesh(core_axis_name="c", subcore_axis_name="s")


_EPW = _BPW * 3


@functools.partial(
    pl.kernel,
    mesh=_mesh,
    out_type=[
        jax.ShapeDtypeStruct((_ON * 3,), jnp.float32),
        jax.ShapeDtypeStruct((_ON * 3,), jnp.float32),
    ],
    scratch_types=[
        pltpu.VMEM((_EPW,), jnp.int32),
        pltpu.VMEM((_EPW,), jnp.float32),
        pltpu.VMEM((_EPW,), jnp.float32),
        pltpu.SemaphoreType.DMA,
        pltpu.SemaphoreType.DMA,
    ],
)
def _sc_gather(coords_hbm, normals_hbm, idx3_hbm, outc_hbm, outn_hbm,
               idx_v, rc_v, rn_v, sem_c, sem_n):
    wid = lax.axis_index("s") * 2 + lax.axis_index("c")
    base = wid * _EPW
    pltpu.sync_copy(idx3_hbm.at[pl.ds(base, _EPW)], idx_v)
    cpy_c = pltpu.async_copy(coords_hbm.at[idx_v], rc_v, sem_c)
    cpy_n = pltpu.async_copy(normals_hbm.at[idx_v], rn_v, sem_n)
    cpy_c.wait()
    cpy_n.wait()
    pltpu.sync_copy(rc_v, outc_hbm.at[pl.ds(base, _EPW)])
    pltpu.sync_copy(rn_v, outn_hbm.at[pl.ds(base, _EPW)])


def kernel(coords, normals, key):
    k = jax.random.key(key)
    idx = jax.random.choice(k, coords.shape[0], shape=(_ON,), replace=False)
    off_coords = jax.random.uniform(k, shape=(_OFF, 3), minval=-1.0, maxval=1.0)
    idx3 = (idx.astype(jnp.int32)[None, :]
            + jnp.arange(3, dtype=jnp.int32)[:, None] * _N).reshape(-1)
    gc, gn = _sc_gather(coords.T.reshape(-1), normals.T.reshape(-1), idx3)
    gc = gc.reshape(3, _ON).T
    gn = gn.reshape(3, _ON).T
    out_coords = jnp.concatenate([gc, off_coords], axis=0)
    out_normals = jnp.concatenate([gn, jnp.full((_OFF, 3), -1.0, jnp.float32)],
                                  axis=0)
    sdf = jnp.concatenate([jnp.zeros((_ON, 1), jnp.float32),
                           jnp.full((_OFF, 1), -1.0, jnp.float32)], axis=0)
    return out_coords, out_normals, sdf
